# Initial kernel scaffold; baseline (speedup 1.0000x reference)
import jax
import jax.numpy as jnp
from jax.experimental import pallas as pl


def kernel(x_main, x_mod, x_map, csr_idx):
    raise NotImplementedError("write your pallas kernel here")


# speedup vs baseline: 43.4492x; 43.4492x over previous
import functools
import jax
import jax.numpy as jnp
from jax import lax
from jax.experimental import pallas as pl
from jax.experimental.pallas import tpu as pltpu
from jax.experimental.pallas import tpu_sc as plsc

NW = 32
L = 16
CAP = 32768
GCHUNK = 64
N_GROUPS = 100000
GPW = N_GROUPS // NW
NB = (GPW + L - 1) // L
GPAD = ((NB * L + GCHUNK - 1) // GCHUNK) * GCHUNK
NCH = GPAD // GCHUNK
CSR_LEN = NB * L + 8
D_MOD = 16
BIGD = 128


def _sc_body(csr_hbm, vals_hbm, xmod_hbm, out_rows,
             csr_v, win_v, args_v, sub_v, seen_v, big_v, rows_v, sem):
    n_mod = xmod_hbm.shape[0] * (BIGD // D_MOD)
    c = lax.axis_index("c")
    s = lax.axis_index("s")
    wid = s * 2 + c
    g0 = wid * GPW
    a0 = pl.multiple_of((g0 // 8) * 8, 8)
    off = g0 - a0
    pltpu.sync_copy(csr_hbm.at[pl.ds(a0, CSR_LEN)], csr_v)

    iot = lax.iota(jnp.int32, L)
    neg = jnp.float32(jnp.finfo(jnp.float32).min)
    negv = jnp.full((L,), neg, jnp.float32)
    sent = jnp.full((L,), n_mod, jnp.int32)

    def bcast(x):
        return jnp.broadcast_to(x, (L,))

    stv0 = csr_v[pl.ds(off, L)]
    w_init = pl.multiple_of((stv0[0] // 8) * 8, 8)
    pltpu.sync_copy(vals_hbm.at[pl.ds(w_init, CAP)], win_v)

    def batch_body(b, w_cur):
        bb = b * L
        stv = csr_v[pl.ds(off + bb, L)]
        env = csr_v[pl.ds(off + bb + 1, L)]
        env = jnp.where((bcast(bb) + iot) < bcast(GPW), env, stv)
        seenv = (env > stv).astype(jnp.int32)
        argv = jnp.zeros((L,), jnp.int32)

        for j in range(L):
            st = stv[j]
            en = env[j]

            def cond_fn(carry):
                p = carry[0]
                return p < en

            def step_fn(carry):
                p, w, am, ai = carry

                def reload(args):
                    pp, _ = args
                    wn = pl.multiple_of((pp // 8) * 8, 8)
                    pltpu.sync_copy(vals_hbm.at[pl.ds(wn, CAP)], win_v)
                    return wn

                w = lax.cond(p + L > w + CAP, reload, lambda a: a[1],
                             (p, w))
                idx = bcast(p) + iot
                v = plsc.load_gather(win_v, [idx - bcast(w)])
                v = jnp.where(idx < bcast(en), v, negv)
                upd = v > am
                am = jnp.where(upd, v, am)
                ai = jnp.where(upd, idx, ai)
                return (p + L, w, am, ai)

            am0 = jnp.full((L,), neg, jnp.float32)
            _, w_cur, am, ai = lax.while_loop(
                cond_fn, step_fn, (st, w_cur, am0, sent))
            gmax = jnp.max(am)
            cand = jnp.where(am == bcast(gmax), ai, sent)
            arg = jnp.min(cand)
            arg = jnp.where(en > st, arg, 0)
            argv = jnp.where(iot == j, bcast(arg), argv)

        args_v[pl.ds(bb, L)] = argv >> 3
        sub_v[pl.ds(bb, L)] = (argv & 7) * D_MOD
        seen_v[pl.ds(bb, L)] = seenv.astype(jnp.float32)
        return w_cur

    lax.fori_loop(0, NB, batch_body, w_init)

    zl = jnp.zeros((L,), jnp.int32)

    def pad_body(b, _):
        args_v[pl.ds(b * L, L)] = zl
        sub_v[pl.ds(b * L, L)] = zl
        seen_v[pl.ds(b * L, L)] = jnp.zeros((L,), jnp.float32)
        return 0

    lax.fori_loop(NB, GPAD // L, pad_body, 0)

    def gather_body(j, _):
        base = pl.multiple_of(j * GCHUNK, 8)
        pltpu.async_copy(
            xmod_hbm.at[args_v.at[pl.ds(base, GCHUNK)]],
            big_v, sem).wait()
        for b2 in range(GCHUNK // L):
            sov = sub_v[pl.ds(base + b2 * L, L)]
            sfv = seen_v[pl.ds(base + b2 * L, L)]
            for jj in range(L):
                r = b2 * L + jj
                row = big_v[r, pl.ds(sov[jj], D_MOD)]
                rows_v[pl.ds((base + r) * D_MOD, D_MOD)] = (
                    row * bcast(sfv[jj]))
        return 0

    lax.fori_loop(0, NCH, gather_body, 0)

    obase = pl.multiple_of(wid * (GPW * D_MOD), 8)
    pltpu.sync_copy(rows_v.at[pl.ds(0, GPW * D_MOD)],
                    out_rows.at[pl.ds(obase, GPW * D_MOD)])


@functools.partial(
    pl.kernel,
    mesh=plsc.VectorSubcoreMesh(core_axis_name="c", subcore_axis_name="s"),
    compiler_params=pltpu.CompilerParams(needs_layout_passes=False),
    out_type=jax.ShapeDtypeStruct((NW * GPW * D_MOD,), jnp.float32),
    scratch_types=[
        pltpu.VMEM((CSR_LEN,), jnp.int32),
        pltpu.VMEM((CAP,), jnp.float32),
        pltpu.VMEM((GPAD,), jnp.int32),
        pltpu.VMEM((GPAD,), jnp.int32),
        pltpu.VMEM((GPAD,), jnp.float32),
        pltpu.VMEM((GCHUNK, BIGD), jnp.float32),
        pltpu.VMEM((GPAD * D_MOD,), jnp.float32),
        pltpu.SemaphoreType.DMA,
    ],
)
def _sc_pool(csr_hbm, vals_hbm, xmod_hbm, out_rows,
             csr_v, win_v, args_v, sub_v, seen_v, big_v, rows_v, sem):
    _sc_body(csr_hbm, vals_hbm, xmod_hbm, out_rows,
             csr_v, win_v, args_v, sub_v, seen_v, big_v, rows_v, sem)


@jax.jit
def kernel(x_main, x_mod, x_map, csr_idx):
    del x_main
    csr = csr_idx.astype(jnp.int32)
    csr_pad = jnp.concatenate([csr, jnp.zeros((32,), jnp.int32)])
    vals = x_map[:, 0]
    vals_pad = jnp.concatenate(
        [vals, jnp.zeros((CAP + L,), jnp.float32)])
    xmod_big = x_mod.reshape(-1, BIGD)
    out_rows = _sc_pool(csr_pad, vals_pad, xmod_big)
    x_pool = out_rows.reshape(N_GROUPS, D_MOD)
    x_seen = csr_idx[1:] > csr_idx[:-1]
    return (x_pool, x_seen)
